# baseline (speedup 1.0000x reference)
import jax
import jax.numpy as jnp
from jax import lax
from jax.experimental import pallas as pl

_FIELD = 38462
_F = 26
_E = 16
_BT = 2048


def _scale_kernel(x_ref, w_ref, o_ref):
    w = w_ref[...]
    tiled = jnp.concatenate([w] * _F, axis=1)
    col_f = lax.broadcasted_iota(jnp.int32, (_F, _F * _E), 1) // _E
    row_f = lax.broadcasted_iota(jnp.int32, (_F, _F * _E), 0)
    m = jnp.where(col_f == row_f, tiled, 0.0)
    xf = x_ref[...].astype(jnp.float32)
    o_ref[...] = jnp.dot(xf, m, preferred_element_type=jnp.float32)


@jax.jit
def kernel(x, weight):
    B = x.shape[0]
    w26 = lax.slice(weight, (0, 0), ((_F - 1) * _FIELD + 1, _E), (_FIELD, 1))
    out = pl.pallas_call(
        _scale_kernel,
        grid=(B // _BT,),
        in_specs=[
            pl.BlockSpec((_BT, _F), lambda i: (i, 0)),
            pl.BlockSpec((_F, _E), lambda i: (0, 0)),
        ],
        out_specs=pl.BlockSpec((_BT, _F * _E), lambda i: (i, 0)),
        out_shape=jax.ShapeDtypeStruct((B, _F * _E), jnp.float32),
    )(x, w26)
    return out
